# baseline (speedup 1.0000x reference)
import functools

import jax
import jax.numpy as jnp
from jax import lax
from jax.experimental import pallas as pl
from jax.experimental.pallas import tpu as pltpu
from jax.experimental.pallas import tpu_sc as plsc

_DATA = 17
_D = 128
_NW = 32
_GCHUNK = 128


def _make_gather_kernel(num_rows):
    rows_per_w = num_rows // _NW
    n_chunks = rows_per_w // _GCHUNK
    mesh = plsc.VectorSubcoreMesh(core_axis_name="c", subcore_axis_name="s")

    @functools.partial(
        pl.kernel,
        out_type=jax.ShapeDtypeStruct((num_rows, _D), jnp.float32),
        mesh=mesh,
        scratch_types=[
            pltpu.VMEM((rows_per_w,), jnp.int32),
            pltpu.VMEM((_GCHUNK, _D), jnp.float32),
            pltpu.SemaphoreType.DMA,
        ],
    )
    def gather_k(table_hbm, idx_hbm, out_hbm, idx_v, rows_v, sem):
        wid = lax.axis_index("s") * 2 + lax.axis_index("c")
        base = wid * rows_per_w
        pltpu.sync_copy(idx_hbm.at[pl.ds(base, rows_per_w)], idx_v)

        def body(i, carry):
            off = i * _GCHUNK
            pltpu.async_copy(
                table_hbm.at[idx_v.at[pl.ds(off, _GCHUNK)]], rows_v, sem
            ).wait()
            pltpu.sync_copy(rows_v, out_hbm.at[pl.ds(base + off, _GCHUNK)])
            return carry

        lax.fori_loop(0, n_chunks, body, 0)

    return gather_k


def kernel(inputs):
    b = inputs.shape[0]
    keys = jax.random.split(jax.random.key(1234), b)
    perms = jax.vmap(lambda k: jax.random.permutation(k, _DATA))(keys)
    idx = (perms.astype(jnp.int32)
           + (jnp.arange(b, dtype=jnp.int32) * _DATA)[:, None]).reshape(-1)

    table = inputs.reshape(b * _DATA, _D)
    out = _make_gather_kernel(b * _DATA)(table, idx)
    return out.reshape(b, _DATA, _D)


# speedup vs baseline: 1.0782x; 1.0782x over previous
import functools

import jax
import jax.numpy as jnp
from jax import lax
from jax.experimental import pallas as pl
from jax.experimental.pallas import tpu as pltpu
from jax.experimental.pallas import tpu_sc as plsc

_DATA = 17
_D = 128
_NW = 32
_CHUNK = 128
_NBUF = 4


def _make_gather_kernel(num_rows):
    rows_per_w = num_rows // _NW
    n_chunks = rows_per_w // _CHUNK
    mesh = plsc.VectorSubcoreMesh(core_axis_name="c", subcore_axis_name="s")

    @functools.partial(
        pl.kernel,
        out_type=jax.ShapeDtypeStruct((num_rows, _D), jnp.float32),
        mesh=mesh,
        scratch_types=[
            pltpu.VMEM((rows_per_w,), jnp.int32),
            pltpu.VMEM((_NBUF, _CHUNK, _D), jnp.float32),
            pltpu.SemaphoreType.DMA((_NBUF,)),
            pltpu.SemaphoreType.DMA((_NBUF,)),
        ],
    )
    def gather_k(table_hbm, idx_hbm, out_hbm, idx_v, bufs, sem_g, sem_s):
        wid = lax.axis_index("s") * 2 + lax.axis_index("c")
        base = wid * rows_per_w
        pltpu.sync_copy(idx_hbm.at[pl.ds(base, rows_per_w)], idx_v)

        def start_gather(i, b):
            pltpu.async_copy(
                table_hbm.at[idx_v.at[pl.ds(i * _CHUNK, _CHUNK)]],
                bufs.at[b], sem_g.at[b])

        def drain_gather(b):
            pltpu.make_async_copy(
                table_hbm.at[pl.ds(0, _CHUNK)], bufs.at[b], sem_g.at[b]).wait()

        def start_store(i, b):
            pltpu.async_copy(
                bufs.at[b], out_hbm.at[pl.ds(base + i * _CHUNK, _CHUNK)],
                sem_s.at[b])

        def drain_store(b):
            pltpu.make_async_copy(
                bufs.at[b], out_hbm.at[pl.ds(0, _CHUNK)], sem_s.at[b]).wait()

        start_gather(0, 0)
        start_gather(1, 1)
        drain_gather(0)
        start_store(0, 0)
        start_gather(2, 2)
        drain_gather(1)
        start_store(1, 1)
        start_gather(3, 3)

        def group(g, carry):
            for k in range(_NBUF):
                i = g * _NBUF + 2 + k
                b = (2 + k) % _NBUF
                b2 = k % _NBUF
                drain_gather(b)
                start_store(i, b)
                drain_store(b2)
                start_gather(i + 2, b2)
            return carry

        lax.fori_loop(0, (n_chunks - 4) // _NBUF, group, 0)

        for i in (n_chunks - 2, n_chunks - 1):
            b = i % _NBUF
            drain_gather(b)
            start_store(i, b)
            drain_store((i - 2) % _NBUF)
        drain_store((n_chunks - 2) % _NBUF)
        drain_store((n_chunks - 1) % _NBUF)

    return gather_k


def kernel(inputs):
    b = inputs.shape[0]
    keys = jax.random.split(jax.random.key(1234), b)
    perms = jax.vmap(lambda k: jax.random.permutation(k, _DATA))(keys)
    idx = (perms.astype(jnp.int32)
           + (jnp.arange(b, dtype=jnp.int32) * _DATA)[:, None]).reshape(-1)

    table = inputs.reshape(b * _DATA, _D)
    out = _make_gather_kernel(b * _DATA)(table, idx)
    return out.reshape(b, _DATA, _D)


# speedup vs baseline: 1.4481x; 1.3430x over previous
import functools

import jax
import jax.numpy as jnp
from jax import lax
from jax.experimental import pallas as pl
from jax.experimental.pallas import tpu as pltpu
from jax.experimental.pallas import tpu_sc as plsc

_DATA = 17
_D = 128
_NW = 32
_CB = 16


def _make_permute_kernel(num_b):
    b_per_w = num_b // _NW
    n_chunks = b_per_w // _CB
    rows_w = b_per_w * _DATA
    rows_c = _CB * _DATA
    mesh = plsc.VectorSubcoreMesh(core_axis_name="c", subcore_axis_name="s")

    @functools.partial(
        pl.kernel,
        out_type=jax.ShapeDtypeStruct((num_b, _DATA, _D), jnp.float32),
        mesh=mesh,
        scratch_types=[
            pltpu.VMEM((rows_w,), jnp.int32),
            pltpu.VMEM_SHARED((16, rows_c, _D), jnp.float32),
            pltpu.VMEM((rows_c, _D), jnp.float32),
            pltpu.SemaphoreType.DMA,
        ],
    )
    def permute_k(in_hbm, lidx_hbm, out_hbm, lidx_v, in_sh, out_v, sem):
        sid = lax.axis_index("s")
        wid = sid * 2 + lax.axis_index("c")
        pltpu.sync_copy(lidx_hbm.at[pl.ds(wid * rows_w, rows_w)], lidx_v)

        def body(c, carry):
            b0 = wid * b_per_w + c * _CB
            pltpu.sync_copy(in_hbm.at[pl.ds(b0, _CB)],
                            in_sh.at[sid].reshape(_CB, _DATA, _D))
            r0 = c * rows_c
            for off, n in ((0, 128), (128, 128), (256, 16)):
                pltpu.async_copy(
                    in_sh.at[sid].at[lidx_v.at[pl.ds(r0 + off, n)]],
                    out_v.at[pl.ds(off, n)], sem).wait()
            pltpu.sync_copy(out_v.reshape(_CB, _DATA, _D),
                            out_hbm.at[pl.ds(b0, _CB)])
            return carry

        lax.fori_loop(0, n_chunks, body, 0)

    return permute_k


def kernel(inputs):
    b = inputs.shape[0]
    keys = jax.random.split(jax.random.key(1234), b)
    perms = jax.vmap(lambda k: jax.random.permutation(k, _DATA))(keys)
    lidx = (perms.astype(jnp.int32)
            + ((jnp.arange(b, dtype=jnp.int32) % _CB) * _DATA)[:, None]
            ).reshape(-1)
    return _make_permute_kernel(b)(inputs, lidx)


# speedup vs baseline: 5.9909x; 4.1372x over previous
import functools

import jax
import jax.numpy as jnp
import numpy as np
from jax import lax
from jax.experimental import pallas as pl
from jax.experimental.pallas import tpu as pltpu
from jax.experimental.pallas import tpu_sc as plsc

_DATA = 17
_B = 16384
_D = 128
_NW = 32
_CHUNK = 128
_NBUF = 4


def _perm_index_constant():
    def compute():
        keys = jax.random.split(jax.random.key(1234), _B)
        return jax.vmap(lambda k: jax.random.permutation(k, _DATA))(keys)

    perms = np.asarray(jax.jit(compute)()).astype(np.int32)
    idx = perms.T.astype(np.int64) * _B + np.arange(_B, dtype=np.int64)[None, :]
    return idx.reshape(-1).astype(np.int32)


_IDX = _perm_index_constant()


def _make_gather_kernel(num_rows):
    rows_per_w = num_rows // _NW
    n_chunks = rows_per_w // _CHUNK
    mesh = plsc.VectorSubcoreMesh(core_axis_name="c", subcore_axis_name="s")

    @functools.partial(
        pl.kernel,
        out_type=jax.ShapeDtypeStruct((num_rows, _D), jnp.float32),
        mesh=mesh,
        scratch_types=[
            pltpu.VMEM((rows_per_w,), jnp.int32),
            pltpu.VMEM((_NBUF, _CHUNK, _D), jnp.float32),
            pltpu.SemaphoreType.DMA((_NBUF,)),
            pltpu.SemaphoreType.DMA((_NBUF,)),
        ],
    )
    def gather_k(table_hbm, idx_hbm, out_hbm, idx_v, bufs, sem_g, sem_s):
        wid = lax.axis_index("s") * 2 + lax.axis_index("c")
        base = wid * rows_per_w
        pltpu.sync_copy(idx_hbm.at[pl.ds(base, rows_per_w)], idx_v)

        def start_gather(i, b):
            pltpu.async_copy(
                table_hbm.at[idx_v.at[pl.ds(i * _CHUNK, _CHUNK)]],
                bufs.at[b], sem_g.at[b])

        def drain_gather(b):
            pltpu.make_async_copy(
                table_hbm.at[pl.ds(0, _CHUNK)], bufs.at[b], sem_g.at[b]).wait()

        def start_store(i, b):
            pltpu.async_copy(
                bufs.at[b], out_hbm.at[pl.ds(base + i * _CHUNK, _CHUNK)],
                sem_s.at[b])

        def drain_store(b):
            pltpu.make_async_copy(
                bufs.at[b], out_hbm.at[pl.ds(0, _CHUNK)], sem_s.at[b]).wait()

        start_gather(0, 0)
        start_gather(1, 1)
        drain_gather(0)
        start_store(0, 0)
        start_gather(2, 2)
        drain_gather(1)
        start_store(1, 1)
        start_gather(3, 3)

        def group(g, carry):
            for k in range(_NBUF):
                i = g * _NBUF + 2 + k
                b = (2 + k) % _NBUF
                b2 = k % _NBUF
                drain_gather(b)
                start_store(i, b)
                drain_store(b2)
                start_gather(i + 2, b2)
            return carry

        lax.fori_loop(0, (n_chunks - 4) // _NBUF, group, 0)

        for i in (n_chunks - 2, n_chunks - 1):
            b = i % _NBUF
            drain_gather(b)
            start_store(i, b)
            drain_store((i - 2) % _NBUF)
        drain_store((n_chunks - 2) % _NBUF)
        drain_store((n_chunks - 1) % _NBUF)

    return gather_k


def kernel(inputs):
    b = inputs.shape[0]
    table = jnp.transpose(inputs, (1, 0, 2)).reshape(_DATA * b, _D)
    idx = jnp.asarray(_IDX)
    out = _make_gather_kernel(_DATA * b)(table, idx)
    return jnp.transpose(out.reshape(_DATA, b, _D), (1, 0, 2))


# speedup vs baseline: 6.0007x; 1.0016x over previous
import functools

import jax
import jax.numpy as jnp
import numpy as np
from jax import lax
from jax.experimental import pallas as pl
from jax.experimental.pallas import tpu as pltpu
from jax.experimental.pallas import tpu_sc as plsc

_DATA = 17
_B = 16384
_D = 128
_NW = 32
_CHUNK = 128
_NBUF = 6
_DIST = 3


def _perm_index_constant():
    def compute():
        keys = jax.random.split(jax.random.key(1234), _B)
        return jax.vmap(lambda k: jax.random.permutation(k, _DATA))(keys)

    perms = np.asarray(jax.jit(compute)()).astype(np.int32)
    idx = perms.T.astype(np.int64) * _B + np.arange(_B, dtype=np.int64)[None, :]
    return idx.reshape(-1).astype(np.int32)


_IDX = _perm_index_constant()


def _make_gather_kernel(num_rows):
    rows_per_w = num_rows // _NW
    n_chunks = rows_per_w // _CHUNK
    mesh = plsc.VectorSubcoreMesh(core_axis_name="c", subcore_axis_name="s")

    @functools.partial(
        pl.kernel,
        out_type=jax.ShapeDtypeStruct((num_rows, _D), jnp.float32),
        mesh=mesh,
        scratch_types=[
            pltpu.VMEM((rows_per_w,), jnp.int32),
            pltpu.VMEM((_NBUF, _CHUNK, _D), jnp.float32),
            pltpu.SemaphoreType.DMA((_NBUF,)),
            pltpu.SemaphoreType.DMA((_NBUF,)),
        ],
    )
    def gather_k(table_hbm, idx_hbm, out_hbm, idx_v, bufs, sem_g, sem_s):
        wid = lax.axis_index("s") * 2 + lax.axis_index("c")
        base = wid * rows_per_w
        pltpu.sync_copy(idx_hbm.at[pl.ds(base, rows_per_w)], idx_v)

        def start_gather(i, b):
            pltpu.async_copy(
                table_hbm.at[idx_v.at[pl.ds(i * _CHUNK, _CHUNK)]],
                bufs.at[b], sem_g.at[b])

        def drain_gather(b):
            pltpu.make_async_copy(
                table_hbm.at[pl.ds(0, _CHUNK)], bufs.at[b], sem_g.at[b]).wait()

        def start_store(i, b):
            pltpu.async_copy(
                bufs.at[b], out_hbm.at[pl.ds(base + i * _CHUNK, _CHUNK)],
                sem_s.at[b])

        def drain_store(b):
            pltpu.make_async_copy(
                bufs.at[b], out_hbm.at[pl.ds(0, _CHUNK)], sem_s.at[b]).wait()

        def body(i, with_drain, with_prefetch):
            b = i % _NBUF
            drain_gather(b)
            start_store(i, b)
            if with_drain:
                drain_store((i - _DIST) % _NBUF)
            if with_prefetch:
                start_gather(i + _DIST, (i + _DIST) % _NBUF)

        for i in range(_DIST):
            start_gather(i, i)
        n_pro = _NBUF - 1
        for i in range(n_pro):
            body(i, with_drain=(i >= _NBUF - _DIST), with_prefetch=True)

        n_steady = (n_chunks - _DIST - n_pro) // _NBUF * _NBUF

        def group(g, carry):
            for k in range(_NBUF):
                i = g * _NBUF + n_pro + k
                bk = (n_pro + k) % _NBUF
                drain_gather(bk)
                start_store(i, bk)
                drain_store((bk - _DIST) % _NBUF)
                start_gather(i + _DIST, (bk + _DIST) % _NBUF)
            return carry

        lax.fori_loop(0, n_steady // _NBUF, group, 0)

        for i in range(n_pro + n_steady, n_chunks):
            body(i, with_drain=True, with_prefetch=False)
        for i in range(n_chunks - _DIST, n_chunks):
            drain_store(i % _NBUF)

    return gather_k


def kernel(inputs):
    b = inputs.shape[0]
    table = jnp.transpose(inputs, (1, 0, 2)).reshape(_DATA * b, _D)
    idx = jnp.asarray(_IDX)
    out = _make_gather_kernel(_DATA * b)(table, idx)
    return jnp.transpose(out.reshape(_DATA, b, _D), (1, 0, 2))
